# baseline (speedup 1.0000x reference)
import functools

import jax
import jax.numpy as jnp
from jax import lax
from jax.experimental import pallas as pl
from jax.experimental.pallas import tpu as pltpu
from jax.experimental.pallas import tpu_sc as plsc

F32 = jnp.float32
_PREC = lax.Precision.HIGHEST

_NC = 2
_NS = 16
_NW = _NC * _NS
_KB = 80


def _pick_block(n, want):
    b = min(want, n)
    while n % b or b % 8:
        b -= 8 if b % 8 == 0 else (b % 8)
        if b <= 0:
            return n
    return b


def _edge_body(nrbf, cutoff, dr_ref, zs_ref, we_ref, wr1_ref, br1_ref,
               wr2_ref, br2_ref, *s_refs):
    d = dr_ref[...]
    r2 = jnp.sum(d * d, axis=1, keepdims=True)
    r = jnp.sqrt(r2)
    rhat = d / (r + 1e-9)
    be = d.shape[0]
    mu = lax.broadcasted_iota(jnp.int32, (be, nrbf), 1).astype(F32) * (
        cutoff / (nrbf - 1))
    rbf = jnp.exp(-((r - mu) ** 2) / 0.5)
    y = jnp.pi * (jnp.clip(r / cutoff, 0.0, 1.0) - 0.5)
    y2 = y * y
    siny = y * (1.0 + y2 * (-1.0 / 6.0 + y2 * (1.0 / 120.0 + y2 * (
        -1.0 / 5040.0 + y2 * (1.0 / 362880.0)))))
    fcut = 0.5 * (1.0 - siny)
    t1 = jnp.tanh(jnp.dot(rbf, wr1_ref[...], preferred_element_type=F32,
                          precision=_PREC) + br1_ref[...])
    radial = (jnp.dot(t1, wr2_ref[...], preferred_element_type=F32,
                      precision=_PREC) + br2_ref[...]) * fcut
    zsr = zs_ref[...][0]
    ohT = (jnp.broadcast_to(zsr, (128, be))
           == lax.broadcasted_iota(jnp.int32, (128, be), 0)).astype(F32)
    hsend = lax.dot_general(
        ohT, we_ref[...], dimension_numbers=(((0,), (0,)), ((), ())),
        preferred_element_type=F32)
    msg = hsend * radial
    sel = (lax.broadcasted_iota(jnp.int32, (3, 384), 0)
           == lax.broadcasted_iota(jnp.int32, (3, 384), 1) // 128
           ).astype(F32)
    bc = jnp.dot(rhat, sel, preferred_element_type=F32)
    s_refs[0][...] = msg
    for k in range(3):
        s_refs[k + 1][...] = bc[:, 128 * k:128 * (k + 1)] * msg


def _edge(dr3, zs2, We_pad, W_r1, b_r1, W_r2, b_r2, be):
    e = dr3.shape[0]
    nrbf = W_r1.shape[0]
    body = functools.partial(_edge_body, nrbf, 5.0)
    return pl.pallas_call(
        body,
        grid=(e // be,),
        in_specs=[
            pl.BlockSpec((be, 3), lambda i: (i, 0)),
            pl.BlockSpec((1, 1, be), lambda i: (i, 0, 0)),
            pl.BlockSpec((128, 128), lambda i: (0, 0)),
            pl.BlockSpec((nrbf, 64), lambda i: (0, 0)),
            pl.BlockSpec((1, 64), lambda i: (0, 0)),
            pl.BlockSpec((64, 128), lambda i: (0, 0)),
            pl.BlockSpec((1, 128), lambda i: (0, 0)),
        ],
        out_specs=[pl.BlockSpec((be, 128), lambda i: (i, 0))] * 4,
        out_shape=[jax.ShapeDtypeStruct((e, 128), F32)] * 4,
    )(dr3, zs2, We_pad, W_r1, b_r1.reshape(1, 64), W_r2,
      b_r2.reshape(1, 128))


def _node_body(p_ref, z_ref, we_ref, wu_ref, wo1_ref, bo1_ref, wo2_ref,
               out_ref):
    p = p_ref[...]
    ps = jnp.sum(p, axis=0)
    a0 = ps[0]
    inv1 = ps[1] * ps[1] + ps[2] * ps[2] + ps[3] * ps[3]
    z = z_ref[...]
    zc = jnp.broadcast_to(z, (z.shape[0], 128))
    oh = (zc == lax.broadcasted_iota(jnp.int32, (z.shape[0], 128), 1)).astype(F32)
    h = jnp.dot(oh, we_ref[...], preferred_element_type=F32)
    wu = wu_ref[...]
    upd = (jnp.dot(a0, wu[0:128, :], preferred_element_type=F32,
                   precision=_PREC)
           + jnp.dot(inv1, wu[128:256, :], preferred_element_type=F32,
                     precision=_PREC))
    h_new = h + upd
    pre = jnp.dot(h_new, wo1_ref[...], preferred_element_type=F32,
                  precision=_PREC) + bo1_ref[...]
    hidden = pre * jax.nn.sigmoid(pre)
    out_ref[...] = jnp.sum(hidden * wo2_ref[...], axis=1, keepdims=True)


def _node(P, Z2, We_pad, W_upd, W_o1, b_o1, wo2r, bn):
    nsc = P.shape[0]
    n = Z2.shape[0]
    return pl.pallas_call(
        _node_body,
        grid=(n // bn,),
        in_specs=[
            pl.BlockSpec((nsc, 4, bn, 128), lambda i: (0, 0, i, 0)),
            pl.BlockSpec((bn, 1), lambda i: (i, 0)),
            pl.BlockSpec((128, 128), lambda i: (0, 0)),
            pl.BlockSpec((256, 128), lambda i: (0, 0)),
            pl.BlockSpec((128, 128), lambda i: (0, 0)),
            pl.BlockSpec((1, 128), lambda i: (0, 0)),
            pl.BlockSpec((1, 128), lambda i: (0, 0)),
        ],
        out_specs=pl.BlockSpec((bn, 1), lambda i: (i, 0)),
        out_shape=jax.ShapeDtypeStruct((n, 1), F32),
    )(P, Z2, We_pad, W_upd, W_o1, b_o1.reshape(1, 128), wo2r)


def _sc_zgather(Z, sidx3):
    nblk = sidx3.shape[1]
    n = Z.shape[0]
    mesh = plsc.VectorSubcoreMesh(core_axis_name="c", subcore_axis_name="s",
                                  num_cores=_NC, num_subcores=_NS)

    @functools.partial(
        pl.kernel,
        out_type=jax.ShapeDtypeStruct((_NW, nblk, _KB), jnp.int32),
        mesh=mesh,
        compiler_params=pltpu.CompilerParams(needs_layout_passes=False),
        scratch_types=[
            pltpu.VMEM((n,), jnp.int32),
            pltpu.VMEM((nblk, _KB), jnp.int32),
            pltpu.VMEM((nblk, _KB), jnp.int32),
        ],
    )
    def k(z_hbm, sidx_hbm, zs_hbm, z_v, sidx_v, out_v):
        c = lax.axis_index("c")
        s = lax.axis_index("s")
        wid = s * _NC + c
        pltpu.sync_copy(z_hbm, z_v)
        pltpu.sync_copy(sidx_hbm.at[wid], sidx_v)

        def body(j, carry):
            for g in range(_KB // 16):
                idx16 = sidx_v[j, pl.ds(g * 16, 16)]
                out_v[j, pl.ds(g * 16, 16)] = plsc.load_gather(z_v, [idx16])
            return carry

        lax.fori_loop(0, nblk, body, 0)
        pltpu.sync_copy(out_v, zs_hbm.at[wid])

    return k(Z, sidx3)


def _sc_scatter(s_list, ridx3, n):
    nblk = ridx3.shape[1]
    npass = len(s_list)
    n_pad = ((n + _NS * 128 - 1) // (_NS * 128)) * _NS * 128
    rows_per_sub = n_pad // _NS
    zrows = 40
    nz = rows_per_sub // zrows
    mesh = plsc.VectorSubcoreMesh(core_axis_name="c", subcore_axis_name="s",
                                  num_cores=_NC, num_subcores=_NS)

    @functools.partial(
        pl.kernel,
        out_type=jax.ShapeDtypeStruct((_NC, npass, n_pad, 128), F32),
        mesh=mesh,
        scratch_types=[
            pltpu.VMEM((nblk, _KB), jnp.int32),
            pltpu.VMEM((_KB, 128), F32),
            pltpu.VMEM((_KB, 128), F32),
            pltpu.VMEM((zrows, 128), F32),
            pltpu.VMEM_SHARED((n_pad, 128), F32),
            pltpu.SemaphoreType.DMA,
            pltpu.SemaphoreType.DMA,
        ],
    )
    def k(*refs):
        s_hbms = refs[:npass]
        (ridx_hbm, out_hbm, ridx_v, sbuf0, sbuf1, zbuf, acc,
         sem0, sem1) = refs[npass:]
        bufs, sems = (sbuf0, sbuf1), (sem0, sem1)
        c = lax.axis_index("c")
        s = lax.axis_index("s")
        wid = s * _NC + c
        pltpu.sync_copy(ridx_hbm.at[wid], ridx_v)

        zero16 = jnp.zeros((16,), F32)

        def zinit(i, carry):
            zbuf[i // 8, pl.ds((i % 8) * 16, 16)] = zero16
            return carry

        lax.fori_loop(0, zrows * 8, zinit, 0)

        for cpass in range(npass):
            for kk in range(nz):
                pltpu.sync_copy(
                    zbuf, acc.at[pl.ds(s * rows_per_sub + kk * zrows, zrows)])
            plsc.subcore_barrier()

            src = s_hbms[cpass]
            base = wid * nblk * _KB

            def _src(j):
                return src.at[pl.ds(base + j * _KB, _KB)]

            pltpu.async_copy(_src(0), bufs[0], sems[0])
            pltpu.async_copy(_src(1), bufs[1], sems[1])

            @pl.loop(0, (nblk + 1) // 2)
            def _pair(i):
                for b in range(2):
                    jj = i * 2 + b

                    @pl.when(jj < nblk)
                    def _():
                        pltpu.make_async_copy(
                            _src(jj), bufs[b], sems[b]).wait()
                        pltpu.sync_copy(
                            bufs[b], acc.at[ridx_v.at[jj]], add=True)

                        @pl.when(jj + 2 < nblk)
                        def _():
                            pltpu.async_copy(_src(jj + 2), bufs[b], sems[b])
            plsc.subcore_barrier()
            pltpu.sync_copy(
                acc.at[pl.ds(s * rows_per_sub, rows_per_sub)],
                out_hbm.at[c, cpass, pl.ds(s * rows_per_sub, rows_per_sub)])
            plsc.subcore_barrier()

    return k(*s_list, ridx3)


def kernel(dr_vec, Z, neighbor_idxs, W_embed, W_r1, b_r1, W_r2, b_r2,
           W_upd, W_o1, b_o1, W_o2):
    e = dr_vec.shape[0]
    n = Z.shape[0]
    receivers = neighbor_idxs[0]
    senders = neighbor_idxs[1]

    bn = _pick_block(n, 2000)
    be = _pick_block(e, 1280)

    Z2 = Z.reshape(n, 1).astype(jnp.int32)
    We_pad = jnp.zeros((128, 128), F32).at[:W_embed.shape[0]].set(W_embed)

    per_w = e // _NW
    sidx3 = senders.reshape(_NW, per_w // _KB, _KB).astype(jnp.int32)
    ridx3 = receivers.reshape(_NW, per_w // _KB, _KB).astype(jnp.int32)

    zs = _sc_zgather(Z.astype(jnp.int32), sidx3)
    zs2 = zs.reshape(e // be, 1, be)
    s_list = _edge(dr_vec, zs2, We_pad, W_r1, b_r1, W_r2, b_r2, be)

    P = _sc_scatter(s_list, ridx3, n)

    out = _node(P, Z2, We_pad, W_upd, W_o1, b_o1, W_o2.reshape(1, 128), bn)
    return out.reshape(n)


# speedup vs baseline: 1.0743x; 1.0743x over previous
import functools

import jax
import jax.numpy as jnp
from jax import lax
from jax.experimental import pallas as pl
from jax.experimental.pallas import tpu as pltpu
from jax.experimental.pallas import tpu_sc as plsc

F32 = jnp.float32
_PREC = lax.Precision.HIGHEST

_NC = 2
_NS = 16
_NW = _NC * _NS
_KB = 80


def _pick_block(n, want):
    b = min(want, n)
    while n % b or b % 8:
        b -= 8 if b % 8 == 0 else (b % 8)
        if b <= 0:
            return n
    return b


def _edge_body(nrbf, cutoff, dr_ref, zs_ref, we_ref, wr1_ref, br1_ref,
               wr2_ref, br2_ref, *s_refs):
    d = dr_ref[...]
    r2 = jnp.sum(d * d, axis=1, keepdims=True)
    r = jnp.sqrt(r2)
    rhat = d / (r + 1e-9)
    be = d.shape[0]
    mu = lax.broadcasted_iota(jnp.int32, (be, nrbf), 1).astype(F32) * (
        cutoff / (nrbf - 1))
    rbf = jnp.exp(-((r - mu) ** 2) / 0.5)
    y = jnp.pi * (jnp.clip(r / cutoff, 0.0, 1.0) - 0.5)
    y2 = y * y
    siny = y * (1.0 + y2 * (-1.0 / 6.0 + y2 * (1.0 / 120.0 + y2 * (
        -1.0 / 5040.0 + y2 * (1.0 / 362880.0)))))
    fcut = 0.5 * (1.0 - siny)
    t1 = jnp.tanh(jnp.dot(rbf, wr1_ref[...], preferred_element_type=F32,
                          precision=_PREC) + br1_ref[...])
    radial = (jnp.dot(t1, wr2_ref[...], preferred_element_type=F32,
                      precision=_PREC) + br2_ref[...]) * fcut
    zsr = zs_ref[...][0]
    ohT = (jnp.broadcast_to(zsr, (128, be))
           == lax.broadcasted_iota(jnp.int32, (128, be), 0)).astype(F32)
    hsend = lax.dot_general(
        ohT, we_ref[...], dimension_numbers=(((0,), (0,)), ((), ())),
        preferred_element_type=F32)
    msg = hsend * radial
    s_refs[0][...] = msg
    for k in range(3):
        s_refs[k + 1][...] = rhat[:, k:k + 1] * msg


def _edge(dr3, zs2, We_pad, W_r1, b_r1, W_r2, b_r2, be):
    e = dr3.shape[0]
    nrbf = W_r1.shape[0]
    body = functools.partial(_edge_body, nrbf, 5.0)
    return pl.pallas_call(
        body,
        grid=(e // be,),
        in_specs=[
            pl.BlockSpec((be, 3), lambda i: (i, 0)),
            pl.BlockSpec((1, 1, be), lambda i: (i, 0, 0)),
            pl.BlockSpec((128, 128), lambda i: (0, 0)),
            pl.BlockSpec((nrbf, 64), lambda i: (0, 0)),
            pl.BlockSpec((1, 64), lambda i: (0, 0)),
            pl.BlockSpec((64, 128), lambda i: (0, 0)),
            pl.BlockSpec((1, 128), lambda i: (0, 0)),
        ],
        out_specs=[pl.BlockSpec((be, 128), lambda i: (i, 0))] * 4,
        out_shape=[jax.ShapeDtypeStruct((e, 128), F32)] * 4,
    )(dr3, zs2, We_pad, W_r1, b_r1.reshape(1, 64), W_r2,
      b_r2.reshape(1, 128))


def _node_body(p_ref, z_ref, we_ref, wu_ref, wo1_ref, bo1_ref, wo2_ref,
               out_ref):
    p = p_ref[...]
    ps = jnp.sum(p, axis=0)
    a0 = ps[0]
    inv1 = ps[1] * ps[1] + ps[2] * ps[2] + ps[3] * ps[3]
    z = z_ref[...]
    zc = jnp.broadcast_to(z, (z.shape[0], 128))
    oh = (zc == lax.broadcasted_iota(jnp.int32, (z.shape[0], 128), 1)).astype(F32)
    h = jnp.dot(oh, we_ref[...], preferred_element_type=F32)
    wu = wu_ref[...]
    upd = (jnp.dot(a0, wu[0:128, :], preferred_element_type=F32,
                   precision=_PREC)
           + jnp.dot(inv1, wu[128:256, :], preferred_element_type=F32,
                     precision=_PREC))
    h_new = h + upd
    pre = jnp.dot(h_new, wo1_ref[...], preferred_element_type=F32,
                  precision=_PREC) + bo1_ref[...]
    hidden = pre * jax.nn.sigmoid(pre)
    out_ref[...] = jnp.sum(hidden * wo2_ref[...], axis=1, keepdims=True)


def _node(P, Z2, We_pad, W_upd, W_o1, b_o1, wo2r, bn):
    nsc = P.shape[0]
    n = Z2.shape[0]
    return pl.pallas_call(
        _node_body,
        grid=(n // bn,),
        in_specs=[
            pl.BlockSpec((nsc, 4, bn, 128), lambda i: (0, 0, i, 0)),
            pl.BlockSpec((bn, 1), lambda i: (i, 0)),
            pl.BlockSpec((128, 128), lambda i: (0, 0)),
            pl.BlockSpec((256, 128), lambda i: (0, 0)),
            pl.BlockSpec((128, 128), lambda i: (0, 0)),
            pl.BlockSpec((1, 128), lambda i: (0, 0)),
            pl.BlockSpec((1, 128), lambda i: (0, 0)),
        ],
        out_specs=pl.BlockSpec((bn, 1), lambda i: (i, 0)),
        out_shape=jax.ShapeDtypeStruct((n, 1), F32),
    )(P, Z2, We_pad, W_upd, W_o1, b_o1.reshape(1, 128), wo2r)


def _sc_zgather(Z, sidx3):
    nblk = sidx3.shape[1]
    n = Z.shape[0]
    mesh = plsc.VectorSubcoreMesh(core_axis_name="c", subcore_axis_name="s",
                                  num_cores=_NC, num_subcores=_NS)

    @functools.partial(
        pl.kernel,
        out_type=jax.ShapeDtypeStruct((_NW, nblk, _KB), jnp.int32),
        mesh=mesh,
        compiler_params=pltpu.CompilerParams(needs_layout_passes=False),
        scratch_types=[
            pltpu.VMEM((n,), jnp.int32),
            pltpu.VMEM((nblk, _KB), jnp.int32),
            pltpu.VMEM((nblk, _KB), jnp.int32),
        ],
    )
    def k(z_hbm, sidx_hbm, zs_hbm, z_v, sidx_v, out_v):
        c = lax.axis_index("c")
        s = lax.axis_index("s")
        wid = s * _NC + c
        pltpu.sync_copy(z_hbm, z_v)
        pltpu.sync_copy(sidx_hbm.at[wid], sidx_v)

        def body(j, carry):
            for g in range(_KB // 16):
                idx16 = sidx_v[j, pl.ds(g * 16, 16)]
                out_v[j, pl.ds(g * 16, 16)] = plsc.load_gather(z_v, [idx16])
            return carry

        lax.fori_loop(0, nblk, body, 0)
        pltpu.sync_copy(out_v, zs_hbm.at[wid])

    return k(Z, sidx3)


def _sc_scatter(s_list, ridx3, n):
    nblk = ridx3.shape[1]
    npass = len(s_list)
    n_pad = ((n + _NS * 128 - 1) // (_NS * 128)) * _NS * 128
    rows_per_sub = n_pad // _NS
    zrows = 40
    nz = rows_per_sub // zrows
    mesh = plsc.VectorSubcoreMesh(core_axis_name="c", subcore_axis_name="s",
                                  num_cores=_NC, num_subcores=_NS)

    @functools.partial(
        pl.kernel,
        out_type=jax.ShapeDtypeStruct((_NC, npass, n_pad, 128), F32),
        mesh=mesh,
        scratch_types=[
            pltpu.VMEM((nblk, _KB), jnp.int32),
            pltpu.VMEM((_KB, 128), F32),
            pltpu.VMEM((_KB, 128), F32),
            pltpu.VMEM((zrows, 128), F32),
            pltpu.VMEM_SHARED((n_pad, 128), F32),
            pltpu.SemaphoreType.DMA,
            pltpu.SemaphoreType.DMA,
        ],
    )
    def k(*refs):
        s_hbms = refs[:npass]
        (ridx_hbm, out_hbm, ridx_v, sbuf0, sbuf1, zbuf, acc,
         sem0, sem1) = refs[npass:]
        bufs, sems = (sbuf0, sbuf1), (sem0, sem1)
        c = lax.axis_index("c")
        s = lax.axis_index("s")
        wid = s * _NC + c
        pltpu.sync_copy(ridx_hbm.at[wid], ridx_v)

        zero16 = jnp.zeros((16,), F32)

        def zinit(i, carry):
            zbuf[i // 8, pl.ds((i % 8) * 16, 16)] = zero16
            return carry

        lax.fori_loop(0, zrows * 8, zinit, 0)

        for cpass in range(npass):
            for kk in range(nz):
                pltpu.sync_copy(
                    zbuf, acc.at[pl.ds(s * rows_per_sub + kk * zrows, zrows)])
            plsc.subcore_barrier()

            src = s_hbms[cpass]
            base = wid * nblk * _KB

            def _src(j):
                return src.at[pl.ds(base + j * _KB, _KB)]

            pltpu.async_copy(_src(0), bufs[0], sems[0])
            pltpu.async_copy(_src(1), bufs[1], sems[1])

            @pl.loop(0, (nblk + 1) // 2)
            def _pair(i):
                for b in range(2):
                    jj = i * 2 + b

                    @pl.when(jj < nblk)
                    def _():
                        pltpu.make_async_copy(
                            _src(jj), bufs[b], sems[b]).wait()
                        pltpu.sync_copy(
                            bufs[b], acc.at[ridx_v.at[jj]], add=True)

                        @pl.when(jj + 2 < nblk)
                        def _():
                            pltpu.async_copy(_src(jj + 2), bufs[b], sems[b])
            plsc.subcore_barrier()
            pltpu.sync_copy(
                acc.at[pl.ds(s * rows_per_sub, rows_per_sub)],
                out_hbm.at[c, cpass, pl.ds(s * rows_per_sub, rows_per_sub)])
            plsc.subcore_barrier()

    return k(*s_list, ridx3)


def kernel(dr_vec, Z, neighbor_idxs, W_embed, W_r1, b_r1, W_r2, b_r2,
           W_upd, W_o1, b_o1, W_o2):
    e = dr_vec.shape[0]
    n = Z.shape[0]
    receivers = neighbor_idxs[0]
    senders = neighbor_idxs[1]

    bn = _pick_block(n, 2000)
    be = _pick_block(e, 1280)

    Z2 = Z.reshape(n, 1).astype(jnp.int32)
    We_pad = jnp.zeros((128, 128), F32).at[:W_embed.shape[0]].set(W_embed)

    per_w = e // _NW
    sidx3 = senders.reshape(_NW, per_w // _KB, _KB).astype(jnp.int32)
    ridx3 = receivers.reshape(_NW, per_w // _KB, _KB).astype(jnp.int32)

    zs = _sc_zgather(Z.astype(jnp.int32), sidx3)
    zs2 = zs.reshape(e // be, 1, be)
    s_list = _edge(dr_vec, zs2, We_pad, W_r1, b_r1, W_r2, b_r2, be)

    P = _sc_scatter(s_list, ridx3, n)

    out = _node(P, Z2, We_pad, W_upd, W_o1, b_o1, W_o2.reshape(1, 128), bn)
    return out.reshape(n)
